# baseline (speedup 1.0000x reference)
import functools

import jax
import jax.numpy as jnp
from jax import lax
from jax.experimental import pallas as pl
from jax.experimental.pallas import tpu as pltpu
from jax.experimental.pallas import tpu_sc as plsc

N = 10000
E = 320000
NC, NS, L = 2, 16, 16
NW = NC * NS
EPW = E // NW
NPAD = 10240
STRIPE = NPAD // NS
BB = 32
H = BB // L
NBB = EPW // BB
TOFF = NBB * BB
NBUF = 2


def _proj_body(x_ref, w_ref, al_ref, ar_ref, h_ref, el_ref, er_ref):
    h = jnp.dot(x_ref[...], w_ref[...], preferred_element_type=jnp.float32)
    h_ref[...] = h
    el_ref[...] = jnp.sum(h * al_ref[...], axis=1, keepdims=True)
    er_ref[...] = jnp.sum(h * ar_ref[...], axis=1, keepdims=True)


def _proj(x, W, al, ar):
    n, din = x.shape
    dout = W.shape[1]
    grid = n // 1000
    return pl.pallas_call(
        _proj_body,
        grid=(grid,),
        in_specs=[
            pl.BlockSpec((1000, din), lambda i: (i, 0)),
            pl.BlockSpec((din, dout), lambda i: (0, 0)),
            pl.BlockSpec((1, dout), lambda i: (0, 0)),
            pl.BlockSpec((1, dout), lambda i: (0, 0)),
        ],
        out_specs=[
            pl.BlockSpec((1000, dout), lambda i: (i, 0)),
            pl.BlockSpec((1000, 1), lambda i: (i, 0)),
            pl.BlockSpec((1000, 1), lambda i: (i, 0)),
        ],
        out_shape=[
            jax.ShapeDtypeStruct((n, dout), jnp.float32),
            jax.ShapeDtypeStruct((n, 1), jnp.float32),
            jax.ShapeDtypeStruct((n, 1), jnp.float32),
        ],
    )(x, W, al.reshape(1, dout), ar.reshape(1, dout))


def _merge_proj_body(un_ref, b_ref, w_ref, al_ref, ar_ref,
                     h_ref, el_ref, er_ref):
    din = b_ref.shape[1]
    u = un_ref[0] + un_ref[1]
    den = jnp.maximum(u[:, din:din + 1], 1e-16)
    t = u[:, :din] / den + b_ref[...]
    t = jnp.where(t > 0.0, t, jnp.exp(t) - 1.0)
    h = jnp.dot(t, w_ref[...], preferred_element_type=jnp.float32)
    h_ref[...] = h
    el_ref[...] = jnp.sum(h * al_ref[...], axis=1, keepdims=True)
    er_ref[...] = jnp.sum(h * ar_ref[...], axis=1, keepdims=True)


def _merge_proj(un, b, W, al, ar):
    da = un.shape[2]
    din = da - L
    dout = W.shape[1]
    grid = NPAD // 1280
    return pl.pallas_call(
        _merge_proj_body,
        grid=(grid,),
        in_specs=[
            pl.BlockSpec((2, 1280, da), lambda i: (0, i, 0)),
            pl.BlockSpec((1, din), lambda i: (0, 0)),
            pl.BlockSpec((din, dout), lambda i: (0, 0)),
            pl.BlockSpec((1, dout), lambda i: (0, 0)),
            pl.BlockSpec((1, dout), lambda i: (0, 0)),
        ],
        out_specs=[
            pl.BlockSpec((1280, dout), lambda i: (i, 0)),
            pl.BlockSpec((1280, 1), lambda i: (i, 0)),
            pl.BlockSpec((1280, 1), lambda i: (i, 0)),
        ],
        out_shape=[
            jax.ShapeDtypeStruct((NPAD, dout), jnp.float32),
            jax.ShapeDtypeStruct((NPAD, 1), jnp.float32),
            jax.ShapeDtypeStruct((NPAD, 1), jnp.float32),
        ],
    )(un, b.reshape(1, din), W, al.reshape(1, dout), ar.reshape(1, dout))


def _final_body(un_ref, b_ref, out_ref):
    dout = b_ref.shape[1]
    u = un_ref[0] + un_ref[1]
    den = jnp.maximum(u[:, dout:dout + 1], 1e-16)
    out_ref[...] = u[:, :dout] / den + b_ref[...]


def _final(un, b):
    da = un.shape[2]
    dout = da - L
    grid = N // 1000
    return pl.pallas_call(
        _final_body,
        grid=(grid,),
        in_specs=[
            pl.BlockSpec((2, 1000, da), lambda i: (0, i, 0)),
            pl.BlockSpec((1, dout), lambda i: (0, 0)),
        ],
        out_specs=pl.BlockSpec((1000, dout), lambda i: (i, 0)),
        out_shape=jax.ShapeDtypeStruct((N, dout), jnp.float32),
    )(un, b.reshape(1, dout))


def _make_agg(D, tc_tiling=False):
    DA = D + L
    mesh = plsc.VectorSubcoreMesh(core_axis_name="c", subcore_axis_name="s",
                                  num_cores=NC, num_subcores=NS)

    @functools.partial(
        pl.kernel,
        out_type=jax.ShapeDtypeStruct((NC, NPAD, DA), jnp.float32),
        mesh=mesh,
        compiler_params=pltpu.CompilerParams(
            needs_layout_passes=False, use_tc_tiling_on_sc=tc_tiling),
        scratch_types=[
            pltpu.VMEM((EPW,), jnp.int32),
            pltpu.VMEM((EPW,), jnp.int32),
            pltpu.VMEM((NBUF, BB, D), jnp.float32),
            pltpu.VMEM((NBUF, BB, DA), jnp.float32),
            pltpu.VMEM((NBUF, BB), jnp.int32),
            pltpu.VMEM((NBUF, BB), jnp.float32),
            pltpu.VMEM((NBUF, BB), jnp.float32),
            pltpu.VMEM_SHARED((NPAD, DA), jnp.float32),
        ] + [pltpu.SemaphoreType.DMA] * (4 * NBUF),
    )
    def agg(h_hbm, el_hbm, er_hbm, src_hbm, dst_hbm, z2_hbm, un_out,
            src_v, dst_v, rows_in, rows_out, didx, els_buf, ers_buf,
            acc_sh, *sems):
        gsem = sems[:NBUF]
        ssem = sems[NBUF:2 * NBUF]
        lsem = sems[2 * NBUF:3 * NBUF]
        rsem = sems[3 * NBUF:]
        cid = lax.axis_index("c")
        sid = lax.axis_index("s")
        wid = sid * NC + cid
        base = wid * EPW
        stripe = pl.ds(sid * STRIPE, STRIPE)

        pltpu.sync_copy(z2_hbm, acc_sh.at[stripe])
        pltpu.sync_copy(src_hbm.at[pl.ds(base, EPW)], src_v)
        pltpu.sync_copy(dst_hbm.at[pl.ds(base, EPW)], dst_v)

        for b in range(NBUF):
            s_ref = src_v.at[pl.ds(b * BB, BB)]
            d_ref = dst_v.at[pl.ds(b * BB, BB)]
            pltpu.async_copy(h_hbm.at[s_ref], rows_in.at[b], gsem[b])
            pltpu.async_copy(el_hbm.at[s_ref], els_buf.at[b], lsem[b])
            pltpu.async_copy(er_hbm.at[d_ref], ers_buf.at[b], rsem[b])
        plsc.subcore_barrier()

        def process(i, b, first):
            off = i * BB
            s_ref = src_v.at[pl.ds(off, BB)]
            d_ref = dst_v.at[pl.ds(off, BB)]
            pltpu.make_async_copy(el_hbm.at[s_ref], els_buf.at[b],
                                  lsem[b]).wait()
            pltpu.make_async_copy(er_hbm.at[d_ref], ers_buf.at[b],
                                  rsem[b]).wait()
            ee = []
            for hh in range(H):
                z = (els_buf[b, pl.ds(hh * L, L)]
                     + ers_buf[b, pl.ds(hh * L, L)])
                z = jnp.where(z >= 0.0, z, 0.2 * z)
                ee.append(jnp.exp(z))
            can_next = i < NBB - NBUF
            nxt = off + NBUF * BB

            @pl.when(can_next)
            def _issue_scalar_gathers():
                ns_ref = src_v.at[pl.ds(nxt, BB)]
                nd_ref = dst_v.at[pl.ds(nxt, BB)]
                pltpu.async_copy(el_hbm.at[ns_ref], els_buf.at[b], lsem[b])
                pltpu.async_copy(er_hbm.at[nd_ref], ers_buf.at[b], rsem[b])

            pl.when(jnp.logical_not(first))(
                lambda: pltpu.make_async_copy(
                    rows_out.at[b], acc_sh.at[didx.at[b]], ssem[b]).wait())

            for hh in range(H):
                didx[b, pl.ds(hh * L, L)] = dst_v[pl.ds(off + hh * L, L)]
            lane0 = lax.iota(jnp.int32, L) == 0
            pltpu.make_async_copy(h_hbm.at[s_ref], rows_in.at[b],
                                  gsem[b]).wait()
            for j in range(BB):
                s = jnp.broadcast_to(ee[j // L][j % L], (L,))
                for cb in range(D // L):
                    sl = pl.ds(cb * L, L)
                    rows_out[b, j, sl] = rows_in[b, j, sl] * s
                rows_out[b, j, pl.ds(D, L)] = jnp.where(lane0, s, 0.0)

            @pl.when(can_next)
            def _issue_row_gather():
                pltpu.async_copy(h_hbm.at[src_v.at[pl.ds(nxt, BB)]],
                                 rows_in.at[b], gsem[b])

            pltpu.async_copy(rows_out.at[b], acc_sh.at[didx.at[b]],
                             ssem[b], add=True)

        def chunk(it, carry):
            g = it * NBUF
            for b in range(NBUF):
                process(g + b, b, first=(it == 0))
            return carry

        lax.fori_loop(0, NBB // NBUF, chunk, 0)
        for b in range(NBUF):
            pltpu.make_async_copy(rows_out.at[b], acc_sh.at[didx.at[b]],
                                  ssem[b]).wait()

        tidx_s = src_v[pl.ds(TOFF, L)]
        tidx_d = dst_v[pl.ds(TOFF, L)]
        pltpu.async_copy(el_hbm.at[tidx_s], els_buf.at[0, pl.ds(0, L)],
                         lsem[0])
        pltpu.async_copy(er_hbm.at[tidx_d], ers_buf.at[0, pl.ds(0, L)],
                         rsem[0])
        pltpu.async_copy(h_hbm.at[tidx_s], rows_in.at[0, pl.ds(0, L)],
                         gsem[0])
        pltpu.make_async_copy(el_hbm.at[tidx_s],
                              els_buf.at[0, pl.ds(0, L)], lsem[0]).wait()
        pltpu.make_async_copy(er_hbm.at[tidx_d],
                              ers_buf.at[0, pl.ds(0, L)], rsem[0]).wait()
        tz = els_buf[0, pl.ds(0, L)] + ers_buf[0, pl.ds(0, L)]
        tz = jnp.where(tz >= 0.0, tz, 0.2 * tz)
        tee = jnp.exp(tz)
        tlane0 = lax.iota(jnp.int32, L) == 0
        pltpu.make_async_copy(h_hbm.at[tidx_s],
                              rows_in.at[0, pl.ds(0, L)], gsem[0]).wait()
        for j in range(L):
            s = jnp.broadcast_to(tee[j], (L,))
            for cb in range(D // L):
                sl = pl.ds(cb * L, L)
                rows_out[0, j, sl] = rows_in[0, j, sl] * s
            rows_out[0, j, pl.ds(D, L)] = jnp.where(tlane0, s, 0.0)
        pltpu.sync_copy(rows_out.at[0, pl.ds(0, L)], acc_sh.at[tidx_d],
                        add=True)
        plsc.subcore_barrier()

        pltpu.sync_copy(acc_sh.at[stripe], un_out.at[cid, stripe])

    return agg


_agg = {128: _make_agg(128), 64: _make_agg(64)}


def kernel(features, edge_index, W1, attn_l1, attn_r1, b1,
           W2, attn_l2, attn_r2, b2):
    src = edge_index[0].astype(jnp.int32)
    dst = edge_index[1].astype(jnp.int32)
    z144 = jnp.zeros((STRIPE, 128 + L), jnp.float32)
    z80 = jnp.zeros((STRIPE, 64 + L), jnp.float32)

    h1, el1, er1 = _proj(features, W1, attn_l1, attn_r1)
    un1 = _agg[128](h1, el1.reshape(N), er1.reshape(N), src, dst, z144)
    h2, el2, er2 = _merge_proj(un1, b1, W2, attn_l2, attn_r2)
    un2 = _agg[64](h2, el2.reshape(NPAD), er2.reshape(NPAD), src, dst, z80)
    return _final(un2, b2)


# speedup vs baseline: 1.1459x; 1.1459x over previous
import functools

import jax
import jax.numpy as jnp
from jax import lax
from jax.experimental import pallas as pl
from jax.experimental.pallas import tpu as pltpu
from jax.experimental.pallas import tpu_sc as plsc

N = 10000
E = 320000
NC, NS, L = 2, 16, 16
NW = NC * NS
EPW = E // NW
NPAD = 10240
STRIPE = NPAD // NS
BB = 32
H = BB // L
NBB = EPW // BB
TOFF = NBB * BB
NBUF = 3
IDEPTH = 2 * NBUF


def _proj_body(x_ref, w_ref, al_ref, ar_ref, h_ref, el_ref, er_ref):
    h = jnp.dot(x_ref[...], w_ref[...], preferred_element_type=jnp.float32)
    h_ref[...] = h
    el_ref[...] = jnp.sum(h * al_ref[...], axis=1, keepdims=True)
    er_ref[...] = jnp.sum(h * ar_ref[...], axis=1, keepdims=True)


def _proj(x, W, al, ar):
    n, din = x.shape
    dout = W.shape[1]
    grid = n // 1000
    return pl.pallas_call(
        _proj_body,
        grid=(grid,),
        in_specs=[
            pl.BlockSpec((1000, din), lambda i: (i, 0)),
            pl.BlockSpec((din, dout), lambda i: (0, 0)),
            pl.BlockSpec((1, dout), lambda i: (0, 0)),
            pl.BlockSpec((1, dout), lambda i: (0, 0)),
        ],
        out_specs=[
            pl.BlockSpec((1000, dout), lambda i: (i, 0)),
            pl.BlockSpec((1000, 1), lambda i: (i, 0)),
            pl.BlockSpec((1000, 1), lambda i: (i, 0)),
        ],
        out_shape=[
            jax.ShapeDtypeStruct((n, dout), jnp.float32),
            jax.ShapeDtypeStruct((n, 1), jnp.float32),
            jax.ShapeDtypeStruct((n, 1), jnp.float32),
        ],
    )(x, W, al.reshape(1, dout), ar.reshape(1, dout))


def _merge_proj_body(un_ref, b_ref, w_ref, al_ref, ar_ref,
                     h_ref, el_ref, er_ref):
    din = b_ref.shape[1]
    u = un_ref[0] + un_ref[1]
    den = jnp.maximum(u[:, din:din + 1], 1e-16)
    t = u[:, :din] / den + b_ref[...]
    t = jnp.where(t > 0.0, t, jnp.exp(t) - 1.0)
    h = jnp.dot(t, w_ref[...], preferred_element_type=jnp.float32)
    h_ref[...] = h
    el_ref[...] = jnp.sum(h * al_ref[...], axis=1, keepdims=True)
    er_ref[...] = jnp.sum(h * ar_ref[...], axis=1, keepdims=True)


def _merge_proj(un, b, W, al, ar):
    da = un.shape[2]
    din = da - L
    dout = W.shape[1]
    grid = NPAD // 1280
    return pl.pallas_call(
        _merge_proj_body,
        grid=(grid,),
        in_specs=[
            pl.BlockSpec((2, 1280, da), lambda i: (0, i, 0)),
            pl.BlockSpec((1, din), lambda i: (0, 0)),
            pl.BlockSpec((din, dout), lambda i: (0, 0)),
            pl.BlockSpec((1, dout), lambda i: (0, 0)),
            pl.BlockSpec((1, dout), lambda i: (0, 0)),
        ],
        out_specs=[
            pl.BlockSpec((1280, dout), lambda i: (i, 0)),
            pl.BlockSpec((1280, 1), lambda i: (i, 0)),
            pl.BlockSpec((1280, 1), lambda i: (i, 0)),
        ],
        out_shape=[
            jax.ShapeDtypeStruct((NPAD, dout), jnp.float32),
            jax.ShapeDtypeStruct((NPAD, 1), jnp.float32),
            jax.ShapeDtypeStruct((NPAD, 1), jnp.float32),
        ],
    )(un, b.reshape(1, din), W, al.reshape(1, dout), ar.reshape(1, dout))


def _final_body(un_ref, b_ref, out_ref):
    dout = b_ref.shape[1]
    u = un_ref[0] + un_ref[1]
    den = jnp.maximum(u[:, dout:dout + 1], 1e-16)
    out_ref[...] = u[:, :dout] / den + b_ref[...]


def _final(un, b):
    da = un.shape[2]
    dout = da - L
    grid = N // 1000
    return pl.pallas_call(
        _final_body,
        grid=(grid,),
        in_specs=[
            pl.BlockSpec((2, 1000, da), lambda i: (0, i, 0)),
            pl.BlockSpec((1, dout), lambda i: (0, 0)),
        ],
        out_specs=pl.BlockSpec((1000, dout), lambda i: (i, 0)),
        out_shape=jax.ShapeDtypeStruct((N, dout), jnp.float32),
    )(un, b.reshape(1, dout))


def _make_agg(D, tc_tiling=False):
    DA = D + L
    mesh = plsc.VectorSubcoreMesh(core_axis_name="c", subcore_axis_name="s",
                                  num_cores=NC, num_subcores=NS)

    @functools.partial(
        pl.kernel,
        out_type=jax.ShapeDtypeStruct((NC, NPAD, DA), jnp.float32),
        mesh=mesh,
        compiler_params=pltpu.CompilerParams(
            needs_layout_passes=False, use_tc_tiling_on_sc=tc_tiling),
        scratch_types=[
            pltpu.VMEM((EPW,), jnp.int16),
            pltpu.VMEM((EPW,), jnp.int16),
            pltpu.VMEM((NBUF, BB, D), jnp.float32),
            pltpu.VMEM((NBUF, BB, DA), jnp.float32),
            pltpu.VMEM((IDEPTH, BB), jnp.int32),
            pltpu.VMEM((IDEPTH, BB), jnp.int32),
            pltpu.VMEM((NBUF, BB), jnp.float32),
            pltpu.VMEM((NBUF, BB), jnp.float32),
            pltpu.VMEM((L,), jnp.int32),
            pltpu.VMEM((L,), jnp.int32),
            pltpu.VMEM_SHARED((NPAD, DA), jnp.float32),
        ] + [pltpu.SemaphoreType.DMA] * (4 * NBUF),
    )
    def agg(h_hbm, el_hbm, er_hbm, src_hbm, dst_hbm, tsrc_hbm, tdst_hbm,
            z2_hbm, un_out,
            src_v, dst_v, rows_in, rows_out, sidx, didx, els_buf, ers_buf,
            tsrc_v, tdst_v, acc_sh, *sems):
        gsem = sems[:NBUF]
        ssem = sems[NBUF:2 * NBUF]
        lsem = sems[2 * NBUF:3 * NBUF]
        rsem = sems[3 * NBUF:]
        cid = lax.axis_index("c")
        sid = lax.axis_index("s")
        wid = sid * NC + cid
        base = wid * EPW
        stripe = pl.ds(sid * STRIPE, STRIPE)

        pltpu.sync_copy(z2_hbm, acc_sh.at[stripe])
        pltpu.sync_copy(src_hbm.at[pl.ds(base, EPW)], src_v)
        pltpu.sync_copy(dst_hbm.at[pl.ds(base, EPW)], dst_v)
        pltpu.sync_copy(tsrc_hbm.at[wid], tsrc_v)
        pltpu.sync_copy(tdst_hbm.at[wid], tdst_v)

        def materialize(k, e):
            s16 = src_v[pl.ds(k * BB, BB)]
            s0, s1 = plsc.unpack(s16, format=plsc.PackFormat.INTERLEAVED,
                                 preferred_element_type=jnp.int32)
            sidx[e, pl.ds(0, L)] = s0
            sidx[e, pl.ds(L, L)] = s1
            d16 = dst_v[pl.ds(k * BB, BB)]
            d0, d1 = plsc.unpack(d16, format=plsc.PackFormat.INTERLEAVED,
                                 preferred_element_type=jnp.int32)
            didx[e, pl.ds(0, L)] = d0
            didx[e, pl.ds(L, L)] = d1

        for b in range(NBUF):
            materialize(b, b)
            pltpu.async_copy(h_hbm.at[sidx.at[b]], rows_in.at[b], gsem[b])
            pltpu.async_copy(el_hbm.at[sidx.at[b]], els_buf.at[b], lsem[b])
            pltpu.async_copy(er_hbm.at[didx.at[b]], ers_buf.at[b], rsem[b])
        plsc.subcore_barrier()

        def process(it, b, first):
            i = it * NBUF + b
            par = lax.rem(it, 2)
            ei = b + NBUF * par
            en = b + NBUF * (1 - par)
            pltpu.make_async_copy(el_hbm.at[sidx.at[ei]], els_buf.at[b],
                                  lsem[b]).wait()
            pltpu.make_async_copy(er_hbm.at[didx.at[ei]], ers_buf.at[b],
                                  rsem[b]).wait()
            ee = []
            for hh in range(H):
                z = (els_buf[b, pl.ds(hh * L, L)]
                     + ers_buf[b, pl.ds(hh * L, L)])
                z = jnp.where(z >= 0.0, z, 0.2 * z)
                ee.append(jnp.exp(z))
            can_next = i < NBB - NBUF

            pl.when(jnp.logical_not(first))(
                lambda: pltpu.make_async_copy(
                    rows_out.at[b], acc_sh.at[didx.at[en]], ssem[b]).wait())

            @pl.when(can_next)
            def _issue_scalar_gathers():
                materialize(i + NBUF, en)
                pltpu.async_copy(el_hbm.at[sidx.at[en]], els_buf.at[b],
                                 lsem[b])
                pltpu.async_copy(er_hbm.at[didx.at[en]], ers_buf.at[b],
                                 rsem[b])

            lane0 = lax.iota(jnp.int32, L) == 0
            pltpu.make_async_copy(h_hbm.at[sidx.at[ei]], rows_in.at[b],
                                  gsem[b]).wait()
            for j in range(BB):
                s = jnp.broadcast_to(ee[j // L][j % L], (L,))
                for cb in range(D // L):
                    sl = pl.ds(cb * L, L)
                    rows_out[b, j, sl] = rows_in[b, j, sl] * s
                rows_out[b, j, pl.ds(D, L)] = jnp.where(lane0, s, 0.0)

            @pl.when(can_next)
            def _issue_row_gather():
                pltpu.async_copy(h_hbm.at[sidx.at[en]], rows_in.at[b],
                                 gsem[b])

            pltpu.async_copy(rows_out.at[b], acc_sh.at[didx.at[ei]],
                             ssem[b], add=True)

        def chunk(it, carry):
            for b in range(NBUF):
                process(it, b, first=(it == 0))
            return carry

        lax.fori_loop(0, NBB // NBUF, chunk, 0)
        for b in range(NBUF):
            pltpu.make_async_copy(rows_out.at[b], acc_sh.at[didx.at[b]],
                                  ssem[b]).wait()

        tidx_s = tsrc_v[...]
        tidx_d = tdst_v[...]
        pltpu.async_copy(el_hbm.at[tidx_s], els_buf.at[0, pl.ds(0, L)],
                         lsem[0])
        pltpu.async_copy(er_hbm.at[tidx_d], ers_buf.at[0, pl.ds(0, L)],
                         rsem[0])
        pltpu.async_copy(h_hbm.at[tidx_s], rows_in.at[0, pl.ds(0, L)],
                         gsem[0])
        pltpu.make_async_copy(el_hbm.at[tidx_s],
                              els_buf.at[0, pl.ds(0, L)], lsem[0]).wait()
        pltpu.make_async_copy(er_hbm.at[tidx_d],
                              ers_buf.at[0, pl.ds(0, L)], rsem[0]).wait()
        tz = els_buf[0, pl.ds(0, L)] + ers_buf[0, pl.ds(0, L)]
        tz = jnp.where(tz >= 0.0, tz, 0.2 * tz)
        tee = jnp.exp(tz)
        tlane0 = lax.iota(jnp.int32, L) == 0
        pltpu.make_async_copy(h_hbm.at[tidx_s],
                              rows_in.at[0, pl.ds(0, L)], gsem[0]).wait()
        for j in range(L):
            s = jnp.broadcast_to(tee[j], (L,))
            for cb in range(D // L):
                sl = pl.ds(cb * L, L)
                rows_out[0, j, sl] = rows_in[0, j, sl] * s
            rows_out[0, j, pl.ds(D, L)] = jnp.where(tlane0, s, 0.0)
        pltpu.sync_copy(rows_out.at[0, pl.ds(0, L)], acc_sh.at[tidx_d],
                        add=True)
        plsc.subcore_barrier()

        pltpu.sync_copy(acc_sh.at[stripe], un_out.at[cid, stripe])

    return agg


_agg = {128: _make_agg(128), 64: _make_agg(64)}


def kernel(features, edge_index, W1, attn_l1, attn_r1, b1,
           W2, attn_l2, attn_r2, b2):
    src = edge_index[0].astype(jnp.int32)
    dst = edge_index[1].astype(jnp.int32)
    src16 = src.astype(jnp.int16)
    dst16 = dst.astype(jnp.int16)
    tsrc = src.reshape(NW, EPW)[:, TOFF:]
    tdst = dst.reshape(NW, EPW)[:, TOFF:]
    z144 = jnp.zeros((STRIPE, 128 + L), jnp.float32)
    z80 = jnp.zeros((STRIPE, 64 + L), jnp.float32)

    h1, el1, er1 = _proj(features, W1, attn_l1, attn_r1)
    un1 = _agg[128](h1, el1.reshape(N), er1.reshape(N), src16, dst16,
                    tsrc, tdst, z144)
    h2, el2, er2 = _merge_proj(un1, b1, W2, attn_l2, attn_r2)
    un2 = _agg[64](h2, el2.reshape(NPAD), er2.reshape(NPAD), src16, dst16,
                   tsrc, tdst, z80)
    return _final(un2, b2)


# speedup vs baseline: 1.2184x; 1.0633x over previous
import functools

import jax
import jax.numpy as jnp
from jax import lax
from jax.experimental import pallas as pl
from jax.experimental.pallas import tpu as pltpu
from jax.experimental.pallas import tpu_sc as plsc

N = 10000
E = 320000
NC, NS, L = 2, 16, 16
NW = NC * NS
EPW = E // NW
NPAD = 10240
STRIPE = NPAD // NS
BB = 32
H = BB // L
NBB = EPW // BB
TOFF = NBB * BB
NBUF = 3


def _proj_body(x_ref, w_ref, al_ref, ar_ref, h_ref, el_ref, er_ref):
    h = jnp.dot(x_ref[...], w_ref[...], preferred_element_type=jnp.float32)
    h_ref[...] = h
    el_ref[...] = jnp.sum(h * al_ref[...], axis=1, keepdims=True)
    er_ref[...] = jnp.sum(h * ar_ref[...], axis=1, keepdims=True)


def _proj(x, W, al, ar):
    n, din = x.shape
    dout = W.shape[1]
    grid = n // 1000
    return pl.pallas_call(
        _proj_body,
        grid=(grid,),
        in_specs=[
            pl.BlockSpec((1000, din), lambda i: (i, 0)),
            pl.BlockSpec((din, dout), lambda i: (0, 0)),
            pl.BlockSpec((1, dout), lambda i: (0, 0)),
            pl.BlockSpec((1, dout), lambda i: (0, 0)),
        ],
        out_specs=[
            pl.BlockSpec((1000, dout), lambda i: (i, 0)),
            pl.BlockSpec((1000, 1), lambda i: (i, 0)),
            pl.BlockSpec((1000, 1), lambda i: (i, 0)),
        ],
        out_shape=[
            jax.ShapeDtypeStruct((n, dout), jnp.float32),
            jax.ShapeDtypeStruct((n, 1), jnp.float32),
            jax.ShapeDtypeStruct((n, 1), jnp.float32),
        ],
    )(x, W, al.reshape(1, dout), ar.reshape(1, dout))


def _merge_proj_body(un_ref, den_ref, b_ref, w_ref, al_ref, ar_ref,
                     h_ref, el_ref, er_ref):
    u = un_ref[0] + un_ref[1]
    d = den_ref[0] + den_ref[1]
    t = u / jnp.maximum(d, 1e-16) + b_ref[...]
    t = jnp.where(t > 0.0, t, jnp.exp(t) - 1.0)
    h = jnp.dot(t, w_ref[...], preferred_element_type=jnp.float32)
    h_ref[...] = h
    el_ref[...] = jnp.sum(h * al_ref[...], axis=1, keepdims=True)
    er_ref[...] = jnp.sum(h * ar_ref[...], axis=1, keepdims=True)


def _merge_proj(un, den, b, W, al, ar):
    din = un.shape[2]
    dout = W.shape[1]
    grid = NPAD // 1280
    return pl.pallas_call(
        _merge_proj_body,
        grid=(grid,),
        in_specs=[
            pl.BlockSpec((2, 1280, din), lambda i: (0, i, 0)),
            pl.BlockSpec((2, 1280, 1), lambda i: (0, i, 0)),
            pl.BlockSpec((1, din), lambda i: (0, 0)),
            pl.BlockSpec((din, dout), lambda i: (0, 0)),
            pl.BlockSpec((1, dout), lambda i: (0, 0)),
            pl.BlockSpec((1, dout), lambda i: (0, 0)),
        ],
        out_specs=[
            pl.BlockSpec((1280, dout), lambda i: (i, 0)),
            pl.BlockSpec((1280, 1), lambda i: (i, 0)),
            pl.BlockSpec((1280, 1), lambda i: (i, 0)),
        ],
        out_shape=[
            jax.ShapeDtypeStruct((NPAD, dout), jnp.float32),
            jax.ShapeDtypeStruct((NPAD, 1), jnp.float32),
            jax.ShapeDtypeStruct((NPAD, 1), jnp.float32),
        ],
    )(un, den.reshape(2, NPAD, 1), b.reshape(1, din),
      W, al.reshape(1, dout), ar.reshape(1, dout))


def _final_body(un_ref, den_ref, b_ref, out_ref):
    u = un_ref[0] + un_ref[1]
    d = den_ref[0] + den_ref[1]
    out_ref[...] = u / jnp.maximum(d, 1e-16) + b_ref[...]


def _final(un, den, b):
    dout = un.shape[2]
    grid = N // 1000
    return pl.pallas_call(
        _final_body,
        grid=(grid,),
        in_specs=[
            pl.BlockSpec((2, 1000, dout), lambda i: (0, i, 0)),
            pl.BlockSpec((2, 1000, 1), lambda i: (0, i, 0)),
            pl.BlockSpec((1, dout), lambda i: (0, 0)),
        ],
        out_specs=pl.BlockSpec((1000, dout), lambda i: (i, 0)),
        out_shape=jax.ShapeDtypeStruct((N, dout), jnp.float32),
    )(un, den.reshape(2, NPAD, 1), b.reshape(1, dout))


def _make_agg(D, tc_tiling=False):
    mesh = plsc.VectorSubcoreMesh(core_axis_name="c", subcore_axis_name="s",
                                  num_cores=NC, num_subcores=NS)

    @functools.partial(
        pl.kernel,
        out_type=[
            jax.ShapeDtypeStruct((NC, NPAD, D), jnp.float32),
            jax.ShapeDtypeStruct((NC, NPAD), jnp.float32),
        ],
        mesh=mesh,
        compiler_params=pltpu.CompilerParams(
            needs_layout_passes=False, use_tc_tiling_on_sc=tc_tiling),
        scratch_types=[
            pltpu.VMEM((EPW,), jnp.int32),
            pltpu.VMEM((EPW,), jnp.int32),
            pltpu.VMEM((NBUF, BB, D), jnp.float32),
            pltpu.VMEM((NBUF, BB, D), jnp.float32),
            pltpu.VMEM((NBUF, BB), jnp.float32),
            pltpu.VMEM((NBUF, BB), jnp.int32),
            pltpu.VMEM((NBUF, BB), jnp.float32),
            pltpu.VMEM((NBUF, BB), jnp.float32),
            pltpu.VMEM_SHARED((NPAD, D), jnp.float32),
            pltpu.VMEM_SHARED((NPAD,), jnp.float32),
        ] + [pltpu.SemaphoreType.DMA] * (5 * NBUF),
    )
    def agg(h_hbm, el_hbm, er_hbm, src_hbm, dst_hbm, z2_hbm, z1_hbm,
            un_out, den_out,
            src_v, dst_v, rows_in, rows_out, ee_buf, didx, els_buf, ers_buf,
            acc_sh, den_sh, *sems):
        gsem = sems[:NBUF]
        ssem = sems[NBUF:2 * NBUF]
        dsem = sems[2 * NBUF:3 * NBUF]
        lsem = sems[3 * NBUF:4 * NBUF]
        rsem = sems[4 * NBUF:]
        cid = lax.axis_index("c")
        sid = lax.axis_index("s")
        wid = sid * NC + cid
        base = wid * EPW
        stripe = pl.ds(sid * STRIPE, STRIPE)

        pltpu.sync_copy(z2_hbm, acc_sh.at[stripe])
        pltpu.sync_copy(z1_hbm, den_sh.at[stripe])
        pltpu.sync_copy(src_hbm.at[pl.ds(base, EPW)], src_v)
        pltpu.sync_copy(dst_hbm.at[pl.ds(base, EPW)], dst_v)

        for b in range(NBUF):
            s_ref = src_v.at[pl.ds(b * BB, BB)]
            d_ref = dst_v.at[pl.ds(b * BB, BB)]
            pltpu.async_copy(h_hbm.at[s_ref], rows_in.at[b], gsem[b])
            pltpu.async_copy(el_hbm.at[s_ref], els_buf.at[b], lsem[b])
            pltpu.async_copy(er_hbm.at[d_ref], ers_buf.at[b], rsem[b])
        plsc.subcore_barrier()

        def process(i, b, first):
            off = i * BB
            s_ref = src_v.at[pl.ds(off, BB)]
            d_ref = dst_v.at[pl.ds(off, BB)]
            pltpu.make_async_copy(el_hbm.at[s_ref], els_buf.at[b],
                                  lsem[b]).wait()
            pltpu.make_async_copy(er_hbm.at[d_ref], ers_buf.at[b],
                                  rsem[b]).wait()
            ee = []
            for hh in range(H):
                z = (els_buf[b, pl.ds(hh * L, L)]
                     + ers_buf[b, pl.ds(hh * L, L)])
                z = jnp.where(z >= 0.0, z, 0.2 * z)
                ee.append(jnp.exp(z))
            can_next = i < NBB - NBUF
            nxt = off + NBUF * BB

            @pl.when(can_next)
            def _issue_scalar_gathers():
                ns_ref = src_v.at[pl.ds(nxt, BB)]
                nd_ref = dst_v.at[pl.ds(nxt, BB)]
                pltpu.async_copy(el_hbm.at[ns_ref], els_buf.at[b], lsem[b])
                pltpu.async_copy(er_hbm.at[nd_ref], ers_buf.at[b], rsem[b])

            def drain():
                pltpu.make_async_copy(ee_buf.at[b], den_sh.at[didx.at[b]],
                                      dsem[b]).wait()
                pltpu.make_async_copy(rows_out.at[b], acc_sh.at[didx.at[b]],
                                      ssem[b]).wait()
            pl.when(jnp.logical_not(first))(drain)

            for hh in range(H):
                ee_buf[b, pl.ds(hh * L, L)] = ee[hh]
                didx[b, pl.ds(hh * L, L)] = dst_v[pl.ds(off + hh * L, L)]
            pltpu.async_copy(ee_buf.at[b], den_sh.at[didx.at[b]], dsem[b],
                             add=True)
            pltpu.make_async_copy(h_hbm.at[s_ref], rows_in.at[b],
                                  gsem[b]).wait()
            for j in range(BB):
                s = jnp.broadcast_to(ee[j // L][j % L], (L,))
                for cb in range(D // L):
                    sl = pl.ds(cb * L, L)
                    rows_out[b, j, sl] = rows_in[b, j, sl] * s

            @pl.when(can_next)
            def _issue_row_gather():
                pltpu.async_copy(h_hbm.at[src_v.at[pl.ds(nxt, BB)]],
                                 rows_in.at[b], gsem[b])

            pltpu.async_copy(rows_out.at[b], acc_sh.at[didx.at[b]],
                             ssem[b], add=True)

        def chunk(it, carry):
            g = it * NBUF
            for b in range(NBUF):
                process(g + b, b, first=(it == 0))
            return carry

        lax.fori_loop(0, NBB // NBUF, chunk, 0)
        for b in range(NBUF):
            pltpu.make_async_copy(ee_buf.at[b], den_sh.at[didx.at[b]],
                                  dsem[b]).wait()
            pltpu.make_async_copy(rows_out.at[b], acc_sh.at[didx.at[b]],
                                  ssem[b]).wait()

        tidx_s = src_v[pl.ds(TOFF, L)]
        tidx_d = dst_v[pl.ds(TOFF, L)]
        pltpu.async_copy(el_hbm.at[tidx_s], els_buf.at[0, pl.ds(0, L)],
                         lsem[0])
        pltpu.async_copy(er_hbm.at[tidx_d], ers_buf.at[0, pl.ds(0, L)],
                         rsem[0])
        pltpu.async_copy(h_hbm.at[tidx_s], rows_in.at[0, pl.ds(0, L)],
                         gsem[0])
        pltpu.make_async_copy(el_hbm.at[tidx_s],
                              els_buf.at[0, pl.ds(0, L)], lsem[0]).wait()
        pltpu.make_async_copy(er_hbm.at[tidx_d],
                              ers_buf.at[0, pl.ds(0, L)], rsem[0]).wait()
        tz = els_buf[0, pl.ds(0, L)] + ers_buf[0, pl.ds(0, L)]
        tz = jnp.where(tz >= 0.0, tz, 0.2 * tz)
        tee = jnp.exp(tz)
        ee_buf[0, pl.ds(0, L)] = tee
        pltpu.sync_copy(ee_buf.at[0, pl.ds(0, L)], den_sh.at[tidx_d],
                        add=True)
        pltpu.make_async_copy(h_hbm.at[tidx_s],
                              rows_in.at[0, pl.ds(0, L)], gsem[0]).wait()
        for j in range(L):
            s = jnp.broadcast_to(tee[j], (L,))
            for cb in range(D // L):
                sl = pl.ds(cb * L, L)
                rows_out[0, j, sl] = rows_in[0, j, sl] * s
        pltpu.sync_copy(rows_out.at[0, pl.ds(0, L)], acc_sh.at[tidx_d],
                        add=True)
        plsc.subcore_barrier()

        pltpu.sync_copy(acc_sh.at[stripe], un_out.at[cid, stripe])
        pltpu.sync_copy(den_sh.at[stripe], den_out.at[cid, stripe])

    return agg


_agg = {128: _make_agg(128, tc_tiling=True), 64: _make_agg(64)}


def kernel(features, edge_index, W1, attn_l1, attn_r1, b1,
           W2, attn_l2, attn_r2, b2):
    src = edge_index[0].astype(jnp.int32)
    dst = edge_index[1].astype(jnp.int32)
    z128 = jnp.zeros((STRIPE, 128), jnp.float32)
    z64 = jnp.zeros((STRIPE, 64), jnp.float32)
    z1 = jnp.zeros((STRIPE,), jnp.float32)

    h1, el1, er1 = _proj(features, W1, attn_l1, attn_r1)
    un1, den1 = _agg[128](h1, el1.reshape(N), er1.reshape(N), src, dst,
                          z128, z1)
    h2, el2, er2 = _merge_proj(un1, den1, b1, W2, attn_l2, attn_r2)
    un2, den2 = _agg[64](h2, el2.reshape(NPAD), er2.reshape(NPAD), src, dst,
                         z64, z1)
    return _final(un2, den2, b2)


# speedup vs baseline: 1.2327x; 1.0117x over previous
import functools

import jax
import jax.numpy as jnp
from jax import lax
from jax.experimental import pallas as pl
from jax.experimental.pallas import tpu as pltpu
from jax.experimental.pallas import tpu_sc as plsc

N = 10000
E = 320000
NC, NS, L = 2, 16, 16
NW = NC * NS
EPW = E // NW
NPAD = 10240
STRIPE = NPAD // NS
NBUF = 3


def _proj_body(x_ref, w_ref, al_ref, ar_ref, h_ref, el_ref, er_ref):
    h = jnp.dot(x_ref[...], w_ref[...], preferred_element_type=jnp.float32)
    h_ref[...] = h
    el_ref[...] = jnp.sum(h * al_ref[...], axis=1, keepdims=True)
    er_ref[...] = jnp.sum(h * ar_ref[...], axis=1, keepdims=True)


def _proj(x, W, al, ar):
    n, din = x.shape
    dout = W.shape[1]
    grid = n // 1000
    return pl.pallas_call(
        _proj_body,
        grid=(grid,),
        in_specs=[
            pl.BlockSpec((1000, din), lambda i: (i, 0)),
            pl.BlockSpec((din, dout), lambda i: (0, 0)),
            pl.BlockSpec((1, dout), lambda i: (0, 0)),
            pl.BlockSpec((1, dout), lambda i: (0, 0)),
        ],
        out_specs=[
            pl.BlockSpec((1000, dout), lambda i: (i, 0)),
            pl.BlockSpec((1000, 1), lambda i: (i, 0)),
            pl.BlockSpec((1000, 1), lambda i: (i, 0)),
        ],
        out_shape=[
            jax.ShapeDtypeStruct((n, dout), jnp.float32),
            jax.ShapeDtypeStruct((n, 1), jnp.float32),
            jax.ShapeDtypeStruct((n, 1), jnp.float32),
        ],
    )(x, W, al.reshape(1, dout), ar.reshape(1, dout))


def _merge_proj_body(un_ref, den_ref, b_ref, w_ref, al_ref, ar_ref,
                     h_ref, el_ref, er_ref):
    u = un_ref[0] + un_ref[1]
    d = den_ref[0] + den_ref[1]
    t = u / jnp.maximum(d, 1e-16) + b_ref[...]
    t = jnp.where(t > 0.0, t, jnp.exp(t) - 1.0)
    h = jnp.dot(t, w_ref[...], preferred_element_type=jnp.float32)
    h_ref[...] = h
    el_ref[...] = jnp.sum(h * al_ref[...], axis=1, keepdims=True)
    er_ref[...] = jnp.sum(h * ar_ref[...], axis=1, keepdims=True)


def _merge_proj(un, den, b, W, al, ar):
    din = un.shape[2]
    dout = W.shape[1]
    grid = NPAD // 1280
    return pl.pallas_call(
        _merge_proj_body,
        grid=(grid,),
        in_specs=[
            pl.BlockSpec((2, 1280, din), lambda i: (0, i, 0)),
            pl.BlockSpec((2, 1280, 1), lambda i: (0, i, 0)),
            pl.BlockSpec((1, din), lambda i: (0, 0)),
            pl.BlockSpec((din, dout), lambda i: (0, 0)),
            pl.BlockSpec((1, dout), lambda i: (0, 0)),
            pl.BlockSpec((1, dout), lambda i: (0, 0)),
        ],
        out_specs=[
            pl.BlockSpec((1280, dout), lambda i: (i, 0)),
            pl.BlockSpec((1280, 1), lambda i: (i, 0)),
            pl.BlockSpec((1280, 1), lambda i: (i, 0)),
        ],
        out_shape=[
            jax.ShapeDtypeStruct((NPAD, dout), jnp.float32),
            jax.ShapeDtypeStruct((NPAD, 1), jnp.float32),
            jax.ShapeDtypeStruct((NPAD, 1), jnp.float32),
        ],
    )(un, den.reshape(2, NPAD, 1), b.reshape(1, din),
      W, al.reshape(1, dout), ar.reshape(1, dout))


def _final_body(un_ref, den_ref, b_ref, out_ref):
    u = un_ref[0] + un_ref[1]
    d = den_ref[0] + den_ref[1]
    out_ref[...] = u / jnp.maximum(d, 1e-16) + b_ref[...]


def _final(un, den, b):
    dout = un.shape[2]
    grid = N // 1000
    return pl.pallas_call(
        _final_body,
        grid=(grid,),
        in_specs=[
            pl.BlockSpec((2, 1000, dout), lambda i: (0, i, 0)),
            pl.BlockSpec((2, 1000, 1), lambda i: (0, i, 0)),
            pl.BlockSpec((1, dout), lambda i: (0, 0)),
        ],
        out_specs=pl.BlockSpec((1000, dout), lambda i: (i, 0)),
        out_shape=jax.ShapeDtypeStruct((N, dout), jnp.float32),
    )(un, den.reshape(2, NPAD, 1), b.reshape(1, dout))


def _make_agg(D, BB, tc_tiling=False):
    H = BB // L
    NBB = EPW // BB
    TOFF = NBB * BB
    assert NBB % NBUF == 0 and EPW - TOFF == L
    mesh = plsc.VectorSubcoreMesh(core_axis_name="c", subcore_axis_name="s",
                                  num_cores=NC, num_subcores=NS)

    @functools.partial(
        pl.kernel,
        out_type=[
            jax.ShapeDtypeStruct((NC, NPAD, D), jnp.float32),
            jax.ShapeDtypeStruct((NC, NPAD), jnp.float32),
        ],
        mesh=mesh,
        compiler_params=pltpu.CompilerParams(
            needs_layout_passes=False, use_tc_tiling_on_sc=tc_tiling),
        scratch_types=[
            pltpu.VMEM((EPW,), jnp.int32),
            pltpu.VMEM((EPW,), jnp.int32),
            pltpu.VMEM((NBUF, BB, D), jnp.float32),
            pltpu.VMEM((NBUF, BB, D), jnp.float32),
            pltpu.VMEM((NBUF, BB), jnp.float32),
            pltpu.VMEM((NBUF, BB), jnp.int32),
            pltpu.VMEM((NBUF, BB), jnp.float32),
            pltpu.VMEM((NBUF, BB), jnp.float32),
            pltpu.VMEM_SHARED((NPAD, D), jnp.float32),
            pltpu.VMEM_SHARED((NPAD,), jnp.float32),
        ] + [pltpu.SemaphoreType.DMA] * (5 * NBUF),
    )
    def agg(h_hbm, el_hbm, er_hbm, src_hbm, dst_hbm, z2_hbm, z1_hbm,
            un_out, den_out,
            src_v, dst_v, rows_in, rows_out, ee_buf, didx, els_buf, ers_buf,
            acc_sh, den_sh, *sems):
        gsem = sems[:NBUF]
        ssem = sems[NBUF:2 * NBUF]
        dsem = sems[2 * NBUF:3 * NBUF]
        lsem = sems[3 * NBUF:4 * NBUF]
        rsem = sems[4 * NBUF:]
        cid = lax.axis_index("c")
        sid = lax.axis_index("s")
        wid = sid * NC + cid
        base = wid * EPW
        stripe = pl.ds(sid * STRIPE, STRIPE)

        pltpu.sync_copy(z2_hbm, acc_sh.at[stripe])
        pltpu.sync_copy(z1_hbm, den_sh.at[stripe])
        pltpu.sync_copy(src_hbm.at[pl.ds(base, EPW)], src_v)
        pltpu.sync_copy(dst_hbm.at[pl.ds(base, EPW)], dst_v)

        for b in range(NBUF):
            s_ref = src_v.at[pl.ds(b * BB, BB)]
            d_ref = dst_v.at[pl.ds(b * BB, BB)]
            pltpu.async_copy(h_hbm.at[s_ref], rows_in.at[b], gsem[b])
            pltpu.async_copy(el_hbm.at[s_ref], els_buf.at[b], lsem[b])
            pltpu.async_copy(er_hbm.at[d_ref], ers_buf.at[b], rsem[b])
        plsc.subcore_barrier()

        def process(i, b, first):
            off = i * BB
            s_ref = src_v.at[pl.ds(off, BB)]
            d_ref = dst_v.at[pl.ds(off, BB)]
            pltpu.make_async_copy(el_hbm.at[s_ref], els_buf.at[b],
                                  lsem[b]).wait()
            pltpu.make_async_copy(er_hbm.at[d_ref], ers_buf.at[b],
                                  rsem[b]).wait()
            ee = []
            for hh in range(H):
                z = (els_buf[b, pl.ds(hh * L, L)]
                     + ers_buf[b, pl.ds(hh * L, L)])
                z = jnp.where(z >= 0.0, z, 0.2 * z)
                ee.append(jnp.exp(z))
            can_next = i < NBB - NBUF
            nxt = off + NBUF * BB

            @pl.when(can_next)
            def _issue_scalar_gathers():
                ns_ref = src_v.at[pl.ds(nxt, BB)]
                nd_ref = dst_v.at[pl.ds(nxt, BB)]
                pltpu.async_copy(el_hbm.at[ns_ref], els_buf.at[b], lsem[b])
                pltpu.async_copy(er_hbm.at[nd_ref], ers_buf.at[b], rsem[b])

            def drain():
                pltpu.make_async_copy(ee_buf.at[b], den_sh.at[didx.at[b]],
                                      dsem[b]).wait()
                pltpu.make_async_copy(rows_out.at[b], acc_sh.at[didx.at[b]],
                                      ssem[b]).wait()
            pl.when(jnp.logical_not(first))(drain)

            for hh in range(H):
                ee_buf[b, pl.ds(hh * L, L)] = ee[hh]
                didx[b, pl.ds(hh * L, L)] = dst_v[pl.ds(off + hh * L, L)]
            pltpu.async_copy(ee_buf.at[b], den_sh.at[didx.at[b]], dsem[b],
                             add=True)
            pltpu.make_async_copy(h_hbm.at[s_ref], rows_in.at[b],
                                  gsem[b]).wait()
            for j in range(BB):
                s = jnp.broadcast_to(ee[j // L][j % L], (L,))
                for cb in range(D // L):
                    sl = pl.ds(cb * L, L)
                    rows_out[b, j, sl] = rows_in[b, j, sl] * s

            @pl.when(can_next)
            def _issue_row_gather():
                pltpu.async_copy(h_hbm.at[src_v.at[pl.ds(nxt, BB)]],
                                 rows_in.at[b], gsem[b])

            pltpu.async_copy(rows_out.at[b], acc_sh.at[didx.at[b]],
                             ssem[b], add=True)

        def chunk(it, carry):
            g = it * NBUF
            for b in range(NBUF):
                process(g + b, b, first=(it == 0))
            return carry

        lax.fori_loop(0, NBB // NBUF, chunk, 0)
        for b in range(NBUF):
            pltpu.make_async_copy(ee_buf.at[b], den_sh.at[didx.at[b]],
                                  dsem[b]).wait()
            pltpu.make_async_copy(rows_out.at[b], acc_sh.at[didx.at[b]],
                                  ssem[b]).wait()

        tidx_s = src_v[pl.ds(TOFF, L)]
        tidx_d = dst_v[pl.ds(TOFF, L)]
        pltpu.async_copy(el_hbm.at[tidx_s], els_buf.at[0, pl.ds(0, L)],
                         lsem[0])
        pltpu.async_copy(er_hbm.at[tidx_d], ers_buf.at[0, pl.ds(0, L)],
                         rsem[0])
        pltpu.async_copy(h_hbm.at[tidx_s], rows_in.at[0, pl.ds(0, L)],
                         gsem[0])
        pltpu.make_async_copy(el_hbm.at[tidx_s],
                              els_buf.at[0, pl.ds(0, L)], lsem[0]).wait()
        pltpu.make_async_copy(er_hbm.at[tidx_d],
                              ers_buf.at[0, pl.ds(0, L)], rsem[0]).wait()
        tz = els_buf[0, pl.ds(0, L)] + ers_buf[0, pl.ds(0, L)]
        tz = jnp.where(tz >= 0.0, tz, 0.2 * tz)
        tee = jnp.exp(tz)
        ee_buf[0, pl.ds(0, L)] = tee
        pltpu.sync_copy(ee_buf.at[0, pl.ds(0, L)], den_sh.at[tidx_d],
                        add=True)
        pltpu.make_async_copy(h_hbm.at[tidx_s],
                              rows_in.at[0, pl.ds(0, L)], gsem[0]).wait()
        for j in range(L):
            s = jnp.broadcast_to(tee[j], (L,))
            for cb in range(D // L):
                sl = pl.ds(cb * L, L)
                rows_out[0, j, sl] = rows_in[0, j, sl] * s
        pltpu.sync_copy(rows_out.at[0, pl.ds(0, L)], acc_sh.at[tidx_d],
                        add=True)
        plsc.subcore_barrier()

        pltpu.sync_copy(acc_sh.at[stripe], un_out.at[cid, stripe])
        pltpu.sync_copy(den_sh.at[stripe], den_out.at[cid, stripe])

    return agg


_agg = {128: _make_agg(128, 32, tc_tiling=True), 64: _make_agg(64, 64)}


def kernel(features, edge_index, W1, attn_l1, attn_r1, b1,
           W2, attn_l2, attn_r2, b2):
    src = edge_index[0].astype(jnp.int32)
    dst = edge_index[1].astype(jnp.int32)
    z128 = jnp.zeros((STRIPE, 128), jnp.float32)
    z64 = jnp.zeros((STRIPE, 64), jnp.float32)
    z1 = jnp.zeros((STRIPE,), jnp.float32)

    h1, el1, er1 = _proj(features, W1, attn_l1, attn_r1)
    un1, den1 = _agg[128](h1, el1.reshape(N), er1.reshape(N), src, dst,
                          z128, z1)
    h2, el2, er2 = _merge_proj(un1, den1, b1, W2, attn_l2, attn_r2)
    un2, den2 = _agg[64](h2, el2.reshape(NPAD), er2.reshape(NPAD), src, dst,
                         z64, z1)
    return _final(un2, den2, b2)
